# baseline (speedup 1.0000x reference)
import functools
import math

import numpy as np
import jax
import jax.numpy as jnp
from jax import lax
from jax.experimental import pallas as pl
from jax.experimental.pallas import tpu as pltpu
from jax.experimental.pallas import tpu_sc as plsc

_B, _V, _C = 1, 6, 256
_HID = 256
_NCLS = 10
_DBINS = 64
_KCAM = 100
_GTOPK = 200
_HPAD, _WPAD = 576.0, 1024.0
_LVL_SHAPES = ((72, 128), (36, 64), (18, 32))


def _anchors_np(grid_size=0.05, eps=0.01):
    anchors = []
    for lvl, (h, w) in enumerate(_LVL_SHAPES):
        gy, gx = np.meshgrid(np.arange(h, dtype=np.float32),
                             np.arange(w, dtype=np.float32), indexing='ij')
        grid_xy = np.stack([gx, gy], axis=-1)
        grid_xy = (grid_xy + 0.5) / np.array([w, h], dtype=np.float32)
        wh = np.ones_like(grid_xy) * np.float32(grid_size * (2.0 ** lvl))
        anchors.append(np.concatenate([grid_xy, wh], axis=-1).reshape(-1, 4))
    a = np.concatenate(anchors, axis=0).astype(np.float32)
    valid = np.all((a > eps) & (a < 1.0 - eps), axis=-1, keepdims=True)
    safe = np.clip(a, eps, 1.0 - eps).astype(np.float32)
    a = np.log(safe / (1.0 - safe)).astype(np.float32)
    a = np.where(valid, a, np.float32(10000.0))
    return a, valid.astype(np.float32)


_ANCHORS, _VALID = _anchors_np()
_HW = [h * w for h, w in _LVL_SHAPES]
_LVL_OFF = [0, _HW[0], _HW[0] + _HW[1]]
_SC_W = 12288
_KEY_NEG = int(np.int32(np.float32(-1e8).view(np.int32)) ^ np.int32(0x7FFFFFFF))
_IMIN = -2147483648


def _enc_body(with_ff, n_alias, sc_pad, x_ref, vm_ref, eye_ref, mln_ref, pw_ref, pb_ref,
              encw_ref, encb_ref, g_ref, b_ref, sw_ref, sb_ref,
              gw_ref, gb_ref, bw_ref, bb_ref, lg_ref, lb_ref, *rest):
    outs = rest[n_alias:]
    om_ref, sc_ref = outs[0], outs[1]
    x = x_ref[0]
    del eye_ref
    xt = x.T
    m = jnp.dot(xt, pw_ref[...], preferred_element_type=jnp.float32) + pb_ref[...]
    h = jnp.dot(m, encw_ref[...], preferred_element_type=jnp.float32) + encb_ref[...]
    mu = jnp.mean(h, axis=-1, keepdims=True)
    d = h - mu
    va = jnp.mean(d * d, axis=-1, keepdims=True)
    om = d * lax.rsqrt(va + 1e-5) * g_ref[...] + b_ref[...]
    om_ref[0] = om.astype(jnp.bfloat16)
    cls = jnp.dot(om, sw_ref[...], preferred_element_type=jnp.float32) + sb_ref[...]
    sc = jnp.max(cls, axis=-1)
    sc = jnp.where(vm_ref[...][:, 0] > 0.5, sc, -1e8)
    bits = lax.bitcast_convert_type(sc, jnp.int32)
    key = jnp.where(bits < 0, bits ^ jnp.int32(0x7FFFFFFF), bits)
    if sc_pad:
        t = key.shape[0]
        sc_ref[0, 0, pl.ds(0, t)] = key
        sc_ref[0, 0, pl.ds(t, sc_pad)] = jnp.full((sc_pad,), _KEY_NEG, jnp.int32)
    else:
        sc_ref[0, 0] = key
    if with_ff:
        ff_ref = outs[2]
        mu2 = jnp.mean(xt, axis=-1, keepdims=True)
        d2 = xt - mu2
        va2 = jnp.mean(d2 * d2, axis=-1, keepdims=True)
        xn = d2 * lax.rsqrt(va2 + 1e-5) * lg_ref[...] + lb_ref[...]
        mrow = mln_ref[0]
        gamma = jnp.dot(mrow, gw_ref[...],
                        preferred_element_type=jnp.float32) + gb_ref[...]
        beta = jnp.dot(mrow, bw_ref[...],
                       preferred_element_type=jnp.float32) + bb_ref[...]
        ff_ref[0] = xn * (1.0 + gamma) + beta


def _enc_level(feat, vm, mln, pw, pb, encw, encb, g, b, sw, sb,
               gw, gb, bw, bb, lg, lb, tile, with_ff, blk_off,
               sc_blk=None, sc_off=None, om_in=None, sc_in=None, ff_in=None):
    if sc_blk is None:
        sc_blk, sc_off = tile, blk_off
    hw = feat.shape[-1]
    nt = hw // tile
    grid = (_V, nt)
    full = lambda i, j: (0, 0)
    in_specs = [
        pl.BlockSpec((1, _C, tile), lambda i, j: (i, 0, j)),
        pl.BlockSpec((tile, 1), lambda i, j: (j, 0)),
        pl.BlockSpec((_C, _C), full),
        pl.BlockSpec((1, 1, 14), lambda i, j: (i, 0, 0)),
        pl.BlockSpec((_C, _HID), full),
        pl.BlockSpec((1, _HID), full),
        pl.BlockSpec((_HID, _HID), full),
        pl.BlockSpec((1, _HID), full),
        pl.BlockSpec((1, _HID), full),
        pl.BlockSpec((1, _HID), full),
        pl.BlockSpec((_HID, _NCLS), full),
        pl.BlockSpec((1, _NCLS), full),
        pl.BlockSpec((14, _C), full),
        pl.BlockSpec((1, _C), full),
        pl.BlockSpec((14, _C), full),
        pl.BlockSpec((1, _C), full),
        pl.BlockSpec((1, _C), full),
        pl.BlockSpec((1, _C), full),
    ]
    HW_ALL = _HW[0] + _HW[1] + _HW[2]
    FF_ALL = _HW[0] + _HW[1]
    out_shapes = [
        jax.ShapeDtypeStruct((_V, HW_ALL, _HID), jnp.bfloat16),
        jax.ShapeDtypeStruct((_V, 1, _SC_W), jnp.int32),
    ]
    out_specs = [
        pl.BlockSpec((1, tile, _HID), lambda i, j: (i, blk_off + j, 0)),
        pl.BlockSpec((1, 1, sc_blk), lambda i, j: (i, 0, sc_off + j)),
    ]
    if with_ff:
        out_shapes.append(jax.ShapeDtypeStruct((_V, FF_ALL, _C), jnp.float32))
        out_specs.append(
            pl.BlockSpec((1, tile, _C), lambda i, j: (i, blk_off + j, 0)))
    args = [feat, vm, jnp.eye(_C, dtype=jnp.float32), mln, pw, pb,
            encw, encb, g, b, sw, sb, gw, gb, bw, bb, lg, lb]
    aliases = {}
    nin = len(args)
    for buf, out_idx in ((om_in, 0), (sc_in, 1), (ff_in, 2)):
        if buf is not None:
            in_specs.append(pl.BlockSpec(memory_space=pl.ANY))
            aliases[nin] = out_idx
            args.append(buf)
            nin += 1
    return pl.pallas_call(
        functools.partial(_enc_body, with_ff, len(aliases), sc_blk - tile),
        grid=grid,
        in_specs=in_specs,
        out_specs=out_specs,
        out_shape=out_shapes,
        input_output_aliases=aliases,
    )(*args)


_NCH = _SC_W // 16
_KPAD = 112


def _sc_gather_body(om_hbm, rows_hbm, out_hbm, idx_v, rows_v, sem):
    wid = lax.axis_index("c") * 16 + lax.axis_index("s")

    @pl.when(wid < _GTOPK // 8)
    def _():
        base = wid * 8
        pltpu.sync_copy(rows_hbm.at[pl.ds(base, 8)], idx_v)
        pltpu.async_copy(om_hbm.at[idx_v], rows_v, sem).wait()
        pltpu.sync_copy(rows_v, out_hbm.at[pl.ds(base, 8)])


def _sc_gather(om2d, rows):
    mesh = plsc.VectorSubcoreMesh(core_axis_name="c", subcore_axis_name="s")
    return pl.kernel(
        _sc_gather_body,
        out_type=jax.ShapeDtypeStruct((_GTOPK, _HID // 2), jnp.float32),
        mesh=mesh,
        scratch_types=[
            pltpu.VMEM((8,), jnp.int32),
            pltpu.VMEM((8, _HID // 2), jnp.float32),
            pltpu.SemaphoreType.DMA,
        ],
    )(om2d, rows)


def _query_body(qf_ref, anch_ref, intrf_ref, acoef_ref,
                bw1_ref, bb1_ref, bw2_ref, bb2_ref,
                dwa_ref, dwb_ref, db_ref, dbins_ref,
                cx_ref, cy_ref, cz_ref, dp_ref):
    qf = qf_ref[...]
    h1 = jnp.maximum(
        jnp.dot(qf, bw1_ref[...], preferred_element_type=jnp.float32)
        + bb1_ref[...], 0.0)
    bbox = (jnp.dot(h1, bw2_ref[...], preferred_element_type=jnp.float32)
            + bb2_ref[...] + anch_ref[...])
    rb = jax.nn.sigmoid(bbox)
    logits = (jnp.dot(qf, dwa_ref[...], preferred_element_type=jnp.float32)
              + jnp.dot(intrf_ref[...], dwb_ref[...],
                        preferred_element_type=jnp.float32)
              + db_ref[...])
    lmax = jnp.max(logits, axis=-1, keepdims=True)
    e = jnp.exp(logits - lmax)
    dp = e / jnp.sum(e, axis=-1, keepdims=True)
    dvals = dbins_ref[...]
    u = rb[:, 0:1] * _WPAD
    v = rb[:, 1:2] * _HPAD
    ud = u * dvals
    vd = v * dvals
    A = acoef_ref[...]
    cx_ref[...] = (A[:, 0:1] * ud + A[:, 1:2] * vd
                   + A[:, 2:3] * dvals + A[:, 3:4])
    cy_ref[...] = (A[:, 4:5] * ud + A[:, 5:6] * vd
                   + A[:, 6:7] * dvals + A[:, 7:8])
    cz_ref[...] = (A[:, 8:9] * ud + A[:, 9:10] * vd
                   + A[:, 10:11] * dvals + A[:, 11:12])
    dp_ref[...] = dp


def kernel(feat_s8, feat_s16, feat_s32, intrinsics, extrinsics, extrinsics_inv,
           proj_w0, proj_b0, proj_w1, proj_b1, proj_w2, proj_b2,
           enc_proj_w, enc_proj_b, enc_norm_g, enc_norm_b,
           score_w, score_b, bbox_w1, bbox_b1, bbox_w2, bbox_b2,
           depth_w, depth_b, depth_bins, pc_range,
           sa_ln_g, sa_ln_b, sa_gamma_w, sa_gamma_b, sa_beta_w, sa_beta_b):
    f8 = feat_s8.reshape(_V, _C, _HW[0])
    f16 = feat_s16.reshape(_V, _C, _HW[1])
    f32_ = feat_s32.reshape(_V, _C, _HW[2])
    intr0 = intrinsics[0]
    ext0 = extrinsics[0]
    ext_inv0 = extrinsics_inv[0]

    intr_sc = intr0 / 1000.0
    mln = jnp.concatenate(
        [intr_sc[:, 0, 0:1], intr_sc[:, 1, 1:2],
         ext0[:, :3, :].reshape(_V, 12)], axis=-1).reshape(_V, 1, 14)

    vm8 = jnp.asarray(_VALID[_LVL_OFF[0]:_LVL_OFF[0] + _HW[0]])
    vm16 = jnp.asarray(_VALID[_LVL_OFF[1]:_LVL_OFF[1] + _HW[1]])
    vm32 = jnp.asarray(_VALID[_LVL_OFF[2]:])

    r2 = lambda x: x.reshape(1, -1)
    common = (mln, )
    tail = (enc_proj_w, r2(enc_proj_b), r2(enc_norm_g), r2(enc_norm_b),
            score_w, r2(score_b),
            sa_gamma_w, r2(sa_gamma_b), sa_beta_w, r2(sa_beta_b),
            r2(sa_ln_g), r2(sa_ln_b))

    om8, sc8_, ff8 = _enc_level(f8, vm8, *common, proj_w0, r2(proj_b0),
                                *tail, tile=1536, with_ff=True, blk_off=0)
    om16, sc16_, ff16 = _enc_level(f16, vm16, *common, proj_w1, r2(proj_b1),
                                   *tail, tile=768, with_ff=True, blk_off=12,
                                   om_in=om8, sc_in=sc8_, ff_in=ff8)
    om, sc_ = _enc_level(f32_, vm32, *common, proj_w2, r2(proj_b2),
                         *tail, tile=576, with_ff=False, blk_off=20,
                         sc_blk=768, sc_off=15, om_in=om16, sc_in=sc16_)
    feat_flatten_img = ff16

    keys = sc_[:, 0, :]
    gmax = jnp.max(keys.reshape(_V, _SC_W // 16, 16), axis=-1)
    _, gsel = lax.top_k(gmax, _KCAM)
    cand = jnp.take_along_axis(
        keys.reshape(_V, _SC_W // 16, 16),
        gsel[:, :, None], axis=1).reshape(_V, _KCAM * 16)
    topk_keys, ci2 = lax.top_k(cand, _KCAM)
    gof = jnp.take_along_axis(gsel, ci2 // 16, axis=1)
    topk_ind = gof * 16 + ci2 % 16
    anchors = jnp.asarray(_ANCHORS)

    all_keys = topk_keys.reshape(-1)
    _, g_idx = lax.top_k(all_keys, _GTOPK)
    loc200 = jnp.take(topk_ind.reshape(-1), g_idx)
    cam200 = (g_idx // _KCAM).astype(jnp.int32)
    rows200 = cam200 * (_HW[0] + _HW[1] + _HW[2]) + loc200
    om_f32view = lax.bitcast_convert_type(
        om.reshape(_V * (_HW[0] + _HW[1] + _HW[2]), _HID // 2, 2),
        jnp.float32)
    qf_packed = _sc_gather(om_f32view, rows200)
    qf = lax.bitcast_convert_type(qf_packed, jnp.bfloat16)
    qf = qf.reshape(_GTOPK, _HID).astype(jnp.float32)
    anch_sel = anchors[loc200]

    intr_inv = jnp.linalg.inv(intr0)
    M = jnp.einsum('vij,vjk->vik', ext_inv0, intr_inv)
    lo = pc_range[:3]
    rng = pc_range[3:] - lo
    A = M[:, :3, :] / rng[None, :, None]
    A = A.at[:, :, 3].add(-lo[None, :] / rng[None, :])
    A_q = A.reshape(_V, 12)[cam200]
    intr_feat = (intr0.reshape(_V, 16) * 0.01)[cam200]

    NQ = _GTOPK
    cx, cy, cz, dp = pl.pallas_call(
        _query_body,
        out_shape=[jax.ShapeDtypeStruct((NQ, _DBINS), jnp.float32)] * 4,
    )(qf, anch_sel, intr_feat, A_q,
      bbox_w1, r2(bbox_b1), bbox_w2, r2(bbox_b2),
      depth_w[:_HID], depth_w[_HID:], r2(depth_b), r2(depth_bins))

    dyn_query = jnp.stack([cx, cy, cz, dp], axis=-1)[None]
    query_feats_out = qf[None]

    spatial = jnp.array([[72, 128], [36, 64]], dtype=jnp.int32)
    lidar2img = jnp.einsum('vij,vjk->vik', intr0, ext0)[None]
    return (feat_flatten_img, spatial, lidar2img, dyn_query, query_feats_out)


# speedup vs baseline: 2.9596x; 2.9596x over previous
import functools
import math

import numpy as np
import jax
import jax.numpy as jnp
from jax import lax
from jax.experimental import pallas as pl
from jax.experimental.pallas import tpu as pltpu
from jax.experimental.pallas import tpu_sc as plsc

_B, _V, _C = 1, 6, 256
_HID = 256
_NCLS = 10
_DBINS = 64
_KCAM = 100
_GTOPK = 200
_HPAD, _WPAD = 576.0, 1024.0
_LVL_SHAPES = ((72, 128), (36, 64), (18, 32))


def _anchors_np(grid_size=0.05, eps=0.01):
    anchors = []
    for lvl, (h, w) in enumerate(_LVL_SHAPES):
        gy, gx = np.meshgrid(np.arange(h, dtype=np.float32),
                             np.arange(w, dtype=np.float32), indexing='ij')
        grid_xy = np.stack([gx, gy], axis=-1)
        grid_xy = (grid_xy + 0.5) / np.array([w, h], dtype=np.float32)
        wh = np.ones_like(grid_xy) * np.float32(grid_size * (2.0 ** lvl))
        anchors.append(np.concatenate([grid_xy, wh], axis=-1).reshape(-1, 4))
    a = np.concatenate(anchors, axis=0).astype(np.float32)
    valid = np.all((a > eps) & (a < 1.0 - eps), axis=-1, keepdims=True)
    safe = np.clip(a, eps, 1.0 - eps).astype(np.float32)
    a = np.log(safe / (1.0 - safe)).astype(np.float32)
    a = np.where(valid, a, np.float32(10000.0))
    return a, valid.astype(np.float32)


_ANCHORS, _VALID = _anchors_np()
_HW = [h * w for h, w in _LVL_SHAPES]
_LVL_OFF = [0, _HW[0], _HW[0] + _HW[1]]
_SC_W = 12288
_KEY_NEG = int(np.int32(np.float32(-1e8).view(np.int32)) ^ np.int32(0x7FFFFFFF))
_IMIN = -2147483648


def _enc_body(with_ff, n_alias, sc_pad, x_ref, vm_ref, eye_ref, mln_ref, pw_ref, pb_ref,
              encw_ref, encb_ref, g_ref, b_ref, sw_ref, sb_ref,
              gw_ref, gb_ref, bw_ref, bb_ref, lg_ref, lb_ref, *rest):
    outs = rest[n_alias:]
    om_ref, sc_ref = outs[0], outs[1]
    x = x_ref[0]
    del eye_ref
    xt = x.T
    m = jnp.dot(xt, pw_ref[...], preferred_element_type=jnp.float32) + pb_ref[...]
    h = jnp.dot(m, encw_ref[...], preferred_element_type=jnp.float32) + encb_ref[...]
    mu = jnp.mean(h, axis=-1, keepdims=True)
    d = h - mu
    va = jnp.mean(d * d, axis=-1, keepdims=True)
    om = d * lax.rsqrt(va + 1e-5) * g_ref[...] + b_ref[...]
    om_ref[0] = om
    cls = jnp.dot(om, sw_ref[...], preferred_element_type=jnp.float32) + sb_ref[...]
    sc = jnp.max(cls, axis=-1)
    sc = jnp.where(vm_ref[...][:, 0] > 0.5, sc, -1e8)
    bits = lax.bitcast_convert_type(sc, jnp.int32)
    key = jnp.where(bits < 0, bits ^ jnp.int32(0x7FFFFFFF), bits)
    if sc_pad:
        t = key.shape[0]
        sc_ref[0, 0, pl.ds(0, t)] = key
        sc_ref[0, 0, pl.ds(t, sc_pad)] = jnp.full((sc_pad,), _KEY_NEG, jnp.int32)
    else:
        sc_ref[0, 0] = key
    if with_ff:
        ff_ref = outs[2]
        mu2 = jnp.mean(xt, axis=-1, keepdims=True)
        d2 = xt - mu2
        va2 = jnp.mean(d2 * d2, axis=-1, keepdims=True)
        xn = d2 * lax.rsqrt(va2 + 1e-5) * lg_ref[...] + lb_ref[...]
        mrow = mln_ref[0]
        gamma = jnp.dot(mrow, gw_ref[...],
                        preferred_element_type=jnp.float32) + gb_ref[...]
        beta = jnp.dot(mrow, bw_ref[...],
                       preferred_element_type=jnp.float32) + bb_ref[...]
        ff_ref[0] = xn * (1.0 + gamma) + beta


def _enc_level(feat, vm, mln, pw, pb, encw, encb, g, b, sw, sb,
               gw, gb, bw, bb, lg, lb, tile, with_ff, blk_off,
               sc_blk=None, sc_off=None, om_in=None, sc_in=None, ff_in=None):
    if sc_blk is None:
        sc_blk, sc_off = tile, blk_off
    hw = feat.shape[-1]
    nt = hw // tile
    grid = (_V, nt)
    full = lambda i, j: (0, 0)
    in_specs = [
        pl.BlockSpec((1, _C, tile), lambda i, j: (i, 0, j)),
        pl.BlockSpec((tile, 1), lambda i, j: (j, 0)),
        pl.BlockSpec((_C, _C), full),
        pl.BlockSpec((1, 1, 14), lambda i, j: (i, 0, 0)),
        pl.BlockSpec((_C, _HID), full),
        pl.BlockSpec((1, _HID), full),
        pl.BlockSpec((_HID, _HID), full),
        pl.BlockSpec((1, _HID), full),
        pl.BlockSpec((1, _HID), full),
        pl.BlockSpec((1, _HID), full),
        pl.BlockSpec((_HID, _NCLS), full),
        pl.BlockSpec((1, _NCLS), full),
        pl.BlockSpec((14, _C), full),
        pl.BlockSpec((1, _C), full),
        pl.BlockSpec((14, _C), full),
        pl.BlockSpec((1, _C), full),
        pl.BlockSpec((1, _C), full),
        pl.BlockSpec((1, _C), full),
    ]
    HW_ALL = _HW[0] + _HW[1] + _HW[2]
    FF_ALL = _HW[0] + _HW[1]
    out_shapes = [
        jax.ShapeDtypeStruct((_V, HW_ALL, _HID), jnp.float32),
        jax.ShapeDtypeStruct((_V, 1, _SC_W), jnp.int32),
    ]
    out_specs = [
        pl.BlockSpec((1, tile, _HID), lambda i, j: (i, blk_off + j, 0)),
        pl.BlockSpec((1, 1, sc_blk), lambda i, j: (i, 0, sc_off + j)),
    ]
    if with_ff:
        out_shapes.append(jax.ShapeDtypeStruct((_V, FF_ALL, _C), jnp.float32))
        out_specs.append(
            pl.BlockSpec((1, tile, _C), lambda i, j: (i, blk_off + j, 0)))
    args = [feat, vm, jnp.eye(_C, dtype=jnp.float32), mln, pw, pb,
            encw, encb, g, b, sw, sb, gw, gb, bw, bb, lg, lb]
    aliases = {}
    nin = len(args)
    for buf, out_idx in ((om_in, 0), (sc_in, 1), (ff_in, 2)):
        if buf is not None:
            in_specs.append(pl.BlockSpec(memory_space=pl.ANY))
            aliases[nin] = out_idx
            args.append(buf)
            nin += 1
    return pl.pallas_call(
        functools.partial(_enc_body, with_ff, len(aliases), sc_blk - tile),
        grid=grid,
        in_specs=in_specs,
        out_specs=out_specs,
        out_shape=out_shapes,
        input_output_aliases=aliases,
    )(*args)


_NCH = _SC_W // 16
_KPAD = 112


def _sc_gather_body(om_hbm, rows_hbm, out_hbm, idx_v, rows_v, sem):
    wid = lax.axis_index("c") * 16 + lax.axis_index("s")

    @pl.when(wid < _GTOPK // 8)
    def _():
        base = wid * 8
        pltpu.sync_copy(rows_hbm.at[pl.ds(base, 8)], idx_v)
        pltpu.async_copy(om_hbm.at[idx_v], rows_v, sem).wait()
        pltpu.sync_copy(rows_v, out_hbm.at[pl.ds(base, 8)])


def _sc_gather(om2d, rows):
    mesh = plsc.VectorSubcoreMesh(core_axis_name="c", subcore_axis_name="s")
    return pl.kernel(
        _sc_gather_body,
        out_type=jax.ShapeDtypeStruct((_GTOPK, _HID), jnp.float32),
        mesh=mesh,
        scratch_types=[
            pltpu.VMEM((8,), jnp.int32),
            pltpu.VMEM((8, _HID), jnp.float32),
            pltpu.SemaphoreType.DMA,
        ],
    )(om2d, rows)


def _query_body(qf_ref, anch_ref, intrf_ref, acoef_ref,
                bw1_ref, bb1_ref, bw2_ref, bb2_ref,
                dwa_ref, dwb_ref, db_ref, dbins_ref,
                cx_ref, cy_ref, cz_ref, dp_ref):
    qf = qf_ref[...]
    h1 = jnp.maximum(
        jnp.dot(qf, bw1_ref[...], preferred_element_type=jnp.float32)
        + bb1_ref[...], 0.0)
    bbox = (jnp.dot(h1, bw2_ref[...], preferred_element_type=jnp.float32)
            + bb2_ref[...] + anch_ref[...])
    rb = jax.nn.sigmoid(bbox)
    logits = (jnp.dot(qf, dwa_ref[...], preferred_element_type=jnp.float32)
              + jnp.dot(intrf_ref[...], dwb_ref[...],
                        preferred_element_type=jnp.float32)
              + db_ref[...])
    lmax = jnp.max(logits, axis=-1, keepdims=True)
    e = jnp.exp(logits - lmax)
    dp = e / jnp.sum(e, axis=-1, keepdims=True)
    dvals = dbins_ref[...]
    u = rb[:, 0:1] * _WPAD
    v = rb[:, 1:2] * _HPAD
    ud = u * dvals
    vd = v * dvals
    A = acoef_ref[...]
    cx_ref[...] = (A[:, 0:1] * ud + A[:, 1:2] * vd
                   + A[:, 2:3] * dvals + A[:, 3:4])
    cy_ref[...] = (A[:, 4:5] * ud + A[:, 5:6] * vd
                   + A[:, 6:7] * dvals + A[:, 7:8])
    cz_ref[...] = (A[:, 8:9] * ud + A[:, 9:10] * vd
                   + A[:, 10:11] * dvals + A[:, 11:12])
    dp_ref[...] = dp


def kernel(feat_s8, feat_s16, feat_s32, intrinsics, extrinsics, extrinsics_inv,
           proj_w0, proj_b0, proj_w1, proj_b1, proj_w2, proj_b2,
           enc_proj_w, enc_proj_b, enc_norm_g, enc_norm_b,
           score_w, score_b, bbox_w1, bbox_b1, bbox_w2, bbox_b2,
           depth_w, depth_b, depth_bins, pc_range,
           sa_ln_g, sa_ln_b, sa_gamma_w, sa_gamma_b, sa_beta_w, sa_beta_b):
    f8 = feat_s8.reshape(_V, _C, _HW[0])
    f16 = feat_s16.reshape(_V, _C, _HW[1])
    f32_ = feat_s32.reshape(_V, _C, _HW[2])
    intr0 = intrinsics[0]
    ext0 = extrinsics[0]
    ext_inv0 = extrinsics_inv[0]

    intr_sc = intr0 / 1000.0
    mln = jnp.concatenate(
        [intr_sc[:, 0, 0:1], intr_sc[:, 1, 1:2],
         ext0[:, :3, :].reshape(_V, 12)], axis=-1).reshape(_V, 1, 14)

    vm8 = jnp.asarray(_VALID[_LVL_OFF[0]:_LVL_OFF[0] + _HW[0]])
    vm16 = jnp.asarray(_VALID[_LVL_OFF[1]:_LVL_OFF[1] + _HW[1]])
    vm32 = jnp.asarray(_VALID[_LVL_OFF[2]:])

    r2 = lambda x: x.reshape(1, -1)
    common = (mln, )
    tail = (enc_proj_w, r2(enc_proj_b), r2(enc_norm_g), r2(enc_norm_b),
            score_w, r2(score_b),
            sa_gamma_w, r2(sa_gamma_b), sa_beta_w, r2(sa_beta_b),
            r2(sa_ln_g), r2(sa_ln_b))

    om8, sc8_, ff8 = _enc_level(f8, vm8, *common, proj_w0, r2(proj_b0),
                                *tail, tile=1536, with_ff=True, blk_off=0)
    om16, sc16_, ff16 = _enc_level(f16, vm16, *common, proj_w1, r2(proj_b1),
                                   *tail, tile=768, with_ff=True, blk_off=12,
                                   om_in=om8, sc_in=sc8_, ff_in=ff8)
    om, sc_ = _enc_level(f32_, vm32, *common, proj_w2, r2(proj_b2),
                         *tail, tile=576, with_ff=False, blk_off=20,
                         sc_blk=768, sc_off=15, om_in=om16, sc_in=sc16_)
    feat_flatten_img = ff16

    keys = sc_[:, 0, :]
    gmax = jnp.max(keys.reshape(_V, _SC_W // 16, 16), axis=-1)
    _, gsel = lax.top_k(gmax, _KCAM)
    cand = jnp.take_along_axis(
        keys.reshape(_V, _SC_W // 16, 16),
        gsel[:, :, None], axis=1).reshape(_V, _KCAM * 16)
    topk_keys, ci2 = lax.top_k(cand, _KCAM)
    gof = jnp.take_along_axis(gsel, ci2 // 16, axis=1)
    topk_ind = gof * 16 + ci2 % 16
    anchors = jnp.asarray(_ANCHORS)

    all_keys = topk_keys.reshape(-1)
    _, g_idx = lax.top_k(all_keys, _GTOPK)
    loc200 = jnp.take(topk_ind.reshape(-1), g_idx)
    cam200 = (g_idx // _KCAM).astype(jnp.int32)
    rows200 = cam200 * (_HW[0] + _HW[1] + _HW[2]) + loc200
    qf = _sc_gather(om.reshape(_V * (_HW[0] + _HW[1] + _HW[2]), _HID),
                    rows200)
    anch_sel = anchors[loc200]

    intr_inv = jnp.linalg.inv(intr0)
    M = jnp.einsum('vij,vjk->vik', ext_inv0, intr_inv)
    lo = pc_range[:3]
    rng = pc_range[3:] - lo
    A = M[:, :3, :] / rng[None, :, None]
    A = A.at[:, :, 3].add(-lo[None, :] / rng[None, :])
    A_q = A.reshape(_V, 12)[cam200]
    intr_feat = (intr0.reshape(_V, 16) * 0.01)[cam200]

    NQ = _GTOPK
    cx, cy, cz, dp = pl.pallas_call(
        _query_body,
        out_shape=[jax.ShapeDtypeStruct((NQ, _DBINS), jnp.float32)] * 4,
    )(qf, anch_sel, intr_feat, A_q,
      bbox_w1, r2(bbox_b1), bbox_w2, r2(bbox_b2),
      depth_w[:_HID], depth_w[_HID:], r2(depth_b), r2(depth_bins))

    dyn_query = jnp.stack([cx, cy, cz, dp], axis=-1)[None]
    query_feats_out = qf[None]

    spatial = jnp.array([[72, 128], [36, 64]], dtype=jnp.int32)
    lidar2img = jnp.einsum('vij,vjk->vik', intr0, ext0)[None]
    return (feat_flatten_img, spatial, lidar2img, dyn_query, query_feats_out)


# speedup vs baseline: 2.9886x; 1.0098x over previous
import functools
import math

import numpy as np
import jax
import jax.numpy as jnp
from jax import lax
from jax.experimental import pallas as pl
from jax.experimental.pallas import tpu as pltpu
from jax.experimental.pallas import tpu_sc as plsc

_B, _V, _C = 1, 6, 256
_HID = 256
_NCLS = 10
_DBINS = 64
_KCAM = 100
_GTOPK = 200
_HPAD, _WPAD = 576.0, 1024.0
_LVL_SHAPES = ((72, 128), (36, 64), (18, 32))


def _anchors_np(grid_size=0.05, eps=0.01):
    anchors = []
    for lvl, (h, w) in enumerate(_LVL_SHAPES):
        gy, gx = np.meshgrid(np.arange(h, dtype=np.float32),
                             np.arange(w, dtype=np.float32), indexing='ij')
        grid_xy = np.stack([gx, gy], axis=-1)
        grid_xy = (grid_xy + 0.5) / np.array([w, h], dtype=np.float32)
        wh = np.ones_like(grid_xy) * np.float32(grid_size * (2.0 ** lvl))
        anchors.append(np.concatenate([grid_xy, wh], axis=-1).reshape(-1, 4))
    a = np.concatenate(anchors, axis=0).astype(np.float32)
    valid = np.all((a > eps) & (a < 1.0 - eps), axis=-1, keepdims=True)
    safe = np.clip(a, eps, 1.0 - eps).astype(np.float32)
    a = np.log(safe / (1.0 - safe)).astype(np.float32)
    a = np.where(valid, a, np.float32(10000.0))
    return a, valid.astype(np.float32)


_ANCHORS, _VALID = _anchors_np()
_HW = [h * w for h, w in _LVL_SHAPES]
_LVL_OFF = [0, _HW[0], _HW[0] + _HW[1]]
_SC_W = 12288
_KEY_NEG = int(np.int32(np.float32(-1e8).view(np.int32)) ^ np.int32(0x7FFFFFFF))
_IMIN = -2147483648


def _enc_body(with_ff, n_alias, sc_pad, x_ref, vm_ref, eye_ref, mln_ref, pw_ref, pb_ref,
              encw_ref, encb_ref, g_ref, b_ref, sw_ref, sb_ref,
              gw_ref, gb_ref, bw_ref, bb_ref, lg_ref, lb_ref, *rest):
    outs = rest[n_alias:]
    om_ref, sc_ref = outs[0], outs[1]
    x = x_ref[0]
    del eye_ref, encw_ref, encb_ref
    xt = x.T
    h = jnp.dot(xt, pw_ref[...], preferred_element_type=jnp.float32) + pb_ref[...]
    mu = jnp.mean(h, axis=-1, keepdims=True)
    d = h - mu
    va = jnp.mean(d * d, axis=-1, keepdims=True)
    om = d * lax.rsqrt(va + 1e-5) * g_ref[...] + b_ref[...]
    om_ref[0] = om
    cls = jnp.dot(om, sw_ref[...], preferred_element_type=jnp.float32) + sb_ref[...]
    sc = jnp.max(cls, axis=-1)
    sc = jnp.where(vm_ref[...][:, 0] > 0.5, sc, -1e8)
    bits = lax.bitcast_convert_type(sc, jnp.int32)
    key = jnp.where(bits < 0, bits ^ jnp.int32(0x7FFFFFFF), bits)
    if sc_pad:
        t = key.shape[0]
        sc_ref[0, 0, pl.ds(0, t)] = key
        sc_ref[0, 0, pl.ds(t, sc_pad)] = jnp.full((sc_pad,), _KEY_NEG, jnp.int32)
    else:
        sc_ref[0, 0] = key
    if with_ff:
        ff_ref = outs[2]
        mu2 = jnp.mean(xt, axis=-1, keepdims=True)
        d2 = xt - mu2
        va2 = jnp.mean(d2 * d2, axis=-1, keepdims=True)
        xn = d2 * lax.rsqrt(va2 + 1e-5) * lg_ref[...] + lb_ref[...]
        mrow = mln_ref[0]
        gamma = jnp.dot(mrow, gw_ref[...],
                        preferred_element_type=jnp.float32) + gb_ref[...]
        beta = jnp.dot(mrow, bw_ref[...],
                       preferred_element_type=jnp.float32) + bb_ref[...]
        ff_ref[0] = xn * (1.0 + gamma) + beta


def _enc_level(feat, vm, mln, pw, pb, encw, encb, g, b, sw, sb,
               gw, gb, bw, bb, lg, lb, tile, with_ff, blk_off,
               sc_blk=None, sc_off=None, om_in=None, sc_in=None, ff_in=None):
    if sc_blk is None:
        sc_blk, sc_off = tile, blk_off
    hw = feat.shape[-1]
    nt = hw // tile
    grid = (_V, nt)
    full = lambda i, j: (0, 0)
    in_specs = [
        pl.BlockSpec((1, _C, tile), lambda i, j: (i, 0, j)),
        pl.BlockSpec((tile, 1), lambda i, j: (j, 0)),
        pl.BlockSpec((_C, _C), full),
        pl.BlockSpec((1, 1, 14), lambda i, j: (i, 0, 0)),
        pl.BlockSpec((_C, _HID), full),
        pl.BlockSpec((1, _HID), full),
        pl.BlockSpec((_HID, _HID), full),
        pl.BlockSpec((1, _HID), full),
        pl.BlockSpec((1, _HID), full),
        pl.BlockSpec((1, _HID), full),
        pl.BlockSpec((_HID, _NCLS), full),
        pl.BlockSpec((1, _NCLS), full),
        pl.BlockSpec((14, _C), full),
        pl.BlockSpec((1, _C), full),
        pl.BlockSpec((14, _C), full),
        pl.BlockSpec((1, _C), full),
        pl.BlockSpec((1, _C), full),
        pl.BlockSpec((1, _C), full),
    ]
    HW_ALL = _HW[0] + _HW[1] + _HW[2]
    FF_ALL = _HW[0] + _HW[1]
    out_shapes = [
        jax.ShapeDtypeStruct((_V, HW_ALL, _HID), jnp.float32),
        jax.ShapeDtypeStruct((_V, 1, _SC_W), jnp.int32),
    ]
    out_specs = [
        pl.BlockSpec((1, tile, _HID), lambda i, j: (i, blk_off + j, 0)),
        pl.BlockSpec((1, 1, sc_blk), lambda i, j: (i, 0, sc_off + j)),
    ]
    if with_ff:
        out_shapes.append(jax.ShapeDtypeStruct((_V, FF_ALL, _C), jnp.float32))
        out_specs.append(
            pl.BlockSpec((1, tile, _C), lambda i, j: (i, blk_off + j, 0)))
    args = [feat, vm, jnp.eye(_C, dtype=jnp.float32), mln, pw, pb,
            encw, encb, g, b, sw, sb, gw, gb, bw, bb, lg, lb]
    aliases = {}
    nin = len(args)
    for buf, out_idx in ((om_in, 0), (sc_in, 1), (ff_in, 2)):
        if buf is not None:
            in_specs.append(pl.BlockSpec(memory_space=pl.ANY))
            aliases[nin] = out_idx
            args.append(buf)
            nin += 1
    return pl.pallas_call(
        functools.partial(_enc_body, with_ff, len(aliases), sc_blk - tile),
        grid=grid,
        in_specs=in_specs,
        out_specs=out_specs,
        out_shape=out_shapes,
        input_output_aliases=aliases,
    )(*args)


_NCH = _SC_W // 16
_KPAD = 112


def _sc_gather_body(om_hbm, rows_hbm, out_hbm, idx_v, rows_v, sem):
    wid = lax.axis_index("c") * 16 + lax.axis_index("s")

    @pl.when(wid < _GTOPK // 8)
    def _():
        base = wid * 8
        pltpu.sync_copy(rows_hbm.at[pl.ds(base, 8)], idx_v)
        pltpu.async_copy(om_hbm.at[idx_v], rows_v, sem).wait()
        pltpu.sync_copy(rows_v, out_hbm.at[pl.ds(base, 8)])


def _sc_gather(om2d, rows):
    mesh = plsc.VectorSubcoreMesh(core_axis_name="c", subcore_axis_name="s")
    return pl.kernel(
        _sc_gather_body,
        out_type=jax.ShapeDtypeStruct((_GTOPK, _HID), jnp.float32),
        mesh=mesh,
        scratch_types=[
            pltpu.VMEM((8,), jnp.int32),
            pltpu.VMEM((8, _HID), jnp.float32),
            pltpu.SemaphoreType.DMA,
        ],
    )(om2d, rows)


def _query_body(qf_ref, anch_ref, intrf_ref, acoef_ref,
                bw1_ref, bb1_ref, bw2_ref, bb2_ref,
                dwa_ref, dwb_ref, db_ref, dbins_ref,
                cx_ref, cy_ref, cz_ref, dp_ref):
    qf = qf_ref[...]
    h1 = jnp.maximum(
        jnp.dot(qf, bw1_ref[...], preferred_element_type=jnp.float32)
        + bb1_ref[...], 0.0)
    bbox = (jnp.dot(h1, bw2_ref[...], preferred_element_type=jnp.float32)
            + bb2_ref[...] + anch_ref[...])
    rb = jax.nn.sigmoid(bbox)
    logits = (jnp.dot(qf, dwa_ref[...], preferred_element_type=jnp.float32)
              + jnp.dot(intrf_ref[...], dwb_ref[...],
                        preferred_element_type=jnp.float32)
              + db_ref[...])
    lmax = jnp.max(logits, axis=-1, keepdims=True)
    e = jnp.exp(logits - lmax)
    dp = e / jnp.sum(e, axis=-1, keepdims=True)
    dvals = dbins_ref[...]
    u = rb[:, 0:1] * _WPAD
    v = rb[:, 1:2] * _HPAD
    ud = u * dvals
    vd = v * dvals
    A = acoef_ref[...]
    cx_ref[...] = (A[:, 0:1] * ud + A[:, 1:2] * vd
                   + A[:, 2:3] * dvals + A[:, 3:4])
    cy_ref[...] = (A[:, 4:5] * ud + A[:, 5:6] * vd
                   + A[:, 6:7] * dvals + A[:, 7:8])
    cz_ref[...] = (A[:, 8:9] * ud + A[:, 9:10] * vd
                   + A[:, 10:11] * dvals + A[:, 11:12])
    dp_ref[...] = dp


def kernel(feat_s8, feat_s16, feat_s32, intrinsics, extrinsics, extrinsics_inv,
           proj_w0, proj_b0, proj_w1, proj_b1, proj_w2, proj_b2,
           enc_proj_w, enc_proj_b, enc_norm_g, enc_norm_b,
           score_w, score_b, bbox_w1, bbox_b1, bbox_w2, bbox_b2,
           depth_w, depth_b, depth_bins, pc_range,
           sa_ln_g, sa_ln_b, sa_gamma_w, sa_gamma_b, sa_beta_w, sa_beta_b):
    f8 = feat_s8.reshape(_V, _C, _HW[0])
    f16 = feat_s16.reshape(_V, _C, _HW[1])
    f32_ = feat_s32.reshape(_V, _C, _HW[2])
    intr0 = intrinsics[0]
    ext0 = extrinsics[0]
    ext_inv0 = extrinsics_inv[0]

    intr_sc = intr0 / 1000.0
    mln = jnp.concatenate(
        [intr_sc[:, 0, 0:1], intr_sc[:, 1, 1:2],
         ext0[:, :3, :].reshape(_V, 12)], axis=-1).reshape(_V, 1, 14)

    vm8 = jnp.asarray(_VALID[_LVL_OFF[0]:_LVL_OFF[0] + _HW[0]])
    vm16 = jnp.asarray(_VALID[_LVL_OFF[1]:_LVL_OFF[1] + _HW[1]])
    vm32 = jnp.asarray(_VALID[_LVL_OFF[2]:])

    r2 = lambda x: x.reshape(1, -1)
    common = (mln, )
    tail = (enc_proj_w, r2(enc_proj_b), r2(enc_norm_g), r2(enc_norm_b),
            score_w, r2(score_b),
            sa_gamma_w, r2(sa_gamma_b), sa_beta_w, r2(sa_beta_b),
            r2(sa_ln_g), r2(sa_ln_b))

    w2_0 = proj_w0 @ enc_proj_w
    w2_1 = proj_w1 @ enc_proj_w
    w2_2 = proj_w2 @ enc_proj_w
    b2_0 = r2(proj_b0 @ enc_proj_w + enc_proj_b)
    b2_1 = r2(proj_b1 @ enc_proj_w + enc_proj_b)
    b2_2 = r2(proj_b2 @ enc_proj_w + enc_proj_b)
    om8, sc8_, ff8 = _enc_level(f8, vm8, *common, w2_0, b2_0,
                                *tail, tile=1536, with_ff=True, blk_off=0)
    om16, sc16_, ff16 = _enc_level(f16, vm16, *common, w2_1, b2_1,
                                   *tail, tile=768, with_ff=True, blk_off=12,
                                   om_in=om8, sc_in=sc8_, ff_in=ff8)
    om, sc_ = _enc_level(f32_, vm32, *common, w2_2, b2_2,
                         *tail, tile=576, with_ff=False, blk_off=20,
                         sc_blk=768, sc_off=15, om_in=om16, sc_in=sc16_)
    feat_flatten_img = ff16

    keys = sc_[:, 0, :]
    gmax = jnp.max(keys.reshape(_V, _SC_W // 16, 16), axis=-1)
    _, gsel = lax.top_k(gmax, _KCAM)
    cand = jnp.take_along_axis(
        keys.reshape(_V, _SC_W // 16, 16),
        gsel[:, :, None], axis=1).reshape(_V, _KCAM * 16)
    topk_keys, ci2 = lax.top_k(cand, _KCAM)
    gof = jnp.take_along_axis(gsel, ci2 // 16, axis=1)
    topk_ind = gof * 16 + ci2 % 16
    anchors = jnp.asarray(_ANCHORS)

    all_keys = topk_keys.reshape(-1)
    _, g_idx = lax.top_k(all_keys, _GTOPK)
    loc200 = jnp.take(topk_ind.reshape(-1), g_idx)
    cam200 = (g_idx // _KCAM).astype(jnp.int32)
    rows200 = cam200 * (_HW[0] + _HW[1] + _HW[2]) + loc200
    qf = _sc_gather(om.reshape(_V * (_HW[0] + _HW[1] + _HW[2]), _HID),
                    rows200)
    anch_sel = anchors[loc200]

    intr_inv = jnp.linalg.inv(intr0)
    M = jnp.einsum('vij,vjk->vik', ext_inv0, intr_inv)
    lo = pc_range[:3]
    rng = pc_range[3:] - lo
    A = M[:, :3, :] / rng[None, :, None]
    A = A.at[:, :, 3].add(-lo[None, :] / rng[None, :])
    A_q = A.reshape(_V, 12)[cam200]
    intr_feat = (intr0.reshape(_V, 16) * 0.01)[cam200]

    NQ = _GTOPK
    cx, cy, cz, dp = pl.pallas_call(
        _query_body,
        out_shape=[jax.ShapeDtypeStruct((NQ, _DBINS), jnp.float32)] * 4,
    )(qf, anch_sel, intr_feat, A_q,
      bbox_w1, r2(bbox_b1), bbox_w2, r2(bbox_b2),
      depth_w[:_HID], depth_w[_HID:], r2(depth_b), r2(depth_bins))

    dyn_query = jnp.stack([cx, cy, cz, dp], axis=-1)[None]
    query_feats_out = qf[None]

    spatial = jnp.array([[72, 128], [36, 64]], dtype=jnp.int32)
    lidar2img = jnp.einsum('vij,vjk->vik', intr0, ext0)[None]
    return (feat_flatten_img, spatial, lidar2img, dyn_query, query_feats_out)
